# Initial kernel scaffold; baseline (speedup 1.0000x reference)
import jax
import jax.numpy as jnp
from jax.experimental import pallas as pl


def kernel(x, gate_W, gate_b, W1, b1, W2, b2):
    raise NotImplementedError("write your pallas kernel here")


# speedup vs baseline: 3.1848x; 3.1848x over previous
import functools
import math

import jax
import jax.numpy as jnp
from jax.experimental import pallas as pl
from jax.experimental.pallas import tpu as pltpu

_INTERPRET = False

HID = 1024
INTER = 4096
NE = 8
S = 2048
LANES = 128
BLK = 256
NB = S * 2 // BLK + NE
RPAD = NB * BLK


def _gate_body(x_ref, w_ref, b_ref, gw_ref, i1_ref, i2_ref):
    lg = jnp.dot(x_ref[...], w_ref[...], preferred_element_type=jnp.float32)
    lg = lg + b_ref[...]
    lanes = jax.lax.broadcasted_iota(jnp.int32, lg.shape, 1)
    m1 = jnp.max(lg, axis=1, keepdims=True)
    i1 = jnp.min(jnp.where(lg == m1, lanes, LANES), axis=1, keepdims=True)
    lg2 = jnp.where(lanes == i1, -3e38, lg)
    m2 = jnp.max(lg2, axis=1, keepdims=True)
    i2 = jnp.min(jnp.where(lg2 == m2, lanes, LANES), axis=1, keepdims=True)
    e2 = jnp.exp(m2 - m1)
    w1 = 1.0 / (1.0 + e2)
    w2 = 1.0 - w1
    gw = jnp.where(lanes == i1, w1, 0.0) + jnp.where(lanes == i2, w2, 0.0)
    gw_ref[...] = gw
    i1_ref[...] = jnp.broadcast_to(i1, lg.shape)
    i2_ref[...] = jnp.broadcast_to(i2, lg.shape)


def _gate(x2d, gate_W, gate_b):
    gWp = jnp.pad(gate_W, ((0, 0), (0, LANES - NE)))
    gbp = jnp.concatenate(
        [gate_b, jnp.full((LANES - NE,), -1e30, jnp.float32)]).reshape(1, LANES)
    return pl.pallas_call(
        _gate_body,
        out_shape=(
            jax.ShapeDtypeStruct((S, LANES), jnp.float32),
            jax.ShapeDtypeStruct((S, LANES), jnp.int32),
            jax.ShapeDtypeStruct((S, LANES), jnp.int32),
        ),
        interpret=_INTERPRET,
    )(x2d, gWp, gbp)


IH = INTER // 2


def _gelu(h):
    return 0.5 * h * (1.0 + jax.lax.erf(h * 0.7071067811865476))


def _ffn_body_a(be_ref, xg_ref, w1_ref, b1_ref, w2_ref, out_ref):
    h = jnp.dot(xg_ref[...], w1_ref[0], preferred_element_type=jnp.float32)
    h = _gelu(h + b1_ref[0])
    out_ref[...] = jnp.dot(h, w2_ref[0], preferred_element_type=jnp.float32)


def _ffn_body_b(be_ref, xg_ref, w1_ref, b1_ref, w2_ref, b2_ref, ws_ref,
                part_ref, out_ref):
    h = jnp.dot(xg_ref[...], w1_ref[0], preferred_element_type=jnp.float32)
    h = _gelu(h + b1_ref[0])
    o = jnp.dot(h, w2_ref[0], preferred_element_type=jnp.float32)
    o = o + part_ref[...] + b2_ref[0]
    out_ref[...] = o * ws_ref[...]


def _grouped_ffn(xg, W1, b1, W2, b2, ws, be):
    b1r = b1.reshape(NE, 1, INTER)
    b2r = b2.reshape(NE, 1, HID)
    params = pltpu.CompilerParams(
        dimension_semantics=("arbitrary",),
        vmem_limit_bytes=60 * 1024 * 1024,
    )
    grid_a = pltpu.PrefetchScalarGridSpec(
        num_scalar_prefetch=1,
        grid=(NB,),
        in_specs=[
            pl.BlockSpec((BLK, HID), lambda b, be: (b, 0)),
            pl.BlockSpec((1, HID, IH), lambda b, be: (be[b], 0, 0)),
            pl.BlockSpec((1, 1, IH), lambda b, be: (be[b], 0, 0)),
            pl.BlockSpec((1, IH, HID), lambda b, be: (be[b], 0, 0)),
        ],
        out_specs=pl.BlockSpec((BLK, HID), lambda b, be: (b, 0)),
    )
    part = pl.pallas_call(
        _ffn_body_a,
        grid_spec=grid_a,
        out_shape=jax.ShapeDtypeStruct((RPAD, HID), jnp.float32),
        compiler_params=params,
        interpret=_INTERPRET,
    )(be, xg, W1, b1r, W2)
    grid_b = pltpu.PrefetchScalarGridSpec(
        num_scalar_prefetch=1,
        grid=(NB,),
        in_specs=[
            pl.BlockSpec((BLK, HID), lambda b, be: (b, 0)),
            pl.BlockSpec((1, HID, IH), lambda b, be: (be[b], 0, 1)),
            pl.BlockSpec((1, 1, IH), lambda b, be: (be[b], 0, 1)),
            pl.BlockSpec((1, IH, HID), lambda b, be: (be[b], 1, 0)),
            pl.BlockSpec((1, 1, HID), lambda b, be: (be[b], 0, 0)),
            pl.BlockSpec((BLK, 1), lambda b, be: (b, 0)),
            pl.BlockSpec((BLK, HID), lambda b, be: (b, 0)),
        ],
        out_specs=pl.BlockSpec((BLK, HID), lambda b, be: (b, 0)),
    )
    return pl.pallas_call(
        _ffn_body_b,
        grid_spec=grid_b,
        out_shape=jax.ShapeDtypeStruct((RPAD, HID), jnp.float32),
        compiler_params=params,
        interpret=_INTERPRET,
    )(be, xg, W1, b1r, W2, b2r, ws, part)


def kernel(x, gate_W, gate_b, W1, b1, W2, b2):
    x2d = x[0]
    gwfull, i1b, i2b = _gate(x2d, gate_W, gate_b)
    gw8 = gwfull[:, :NE]
    i1 = i1b[:, 0]
    i2 = i2b[:, 0]

    e_flat = jnp.stack([i1, i2], axis=1).reshape(-1)
    oh = (e_flat[:, None] == jnp.arange(NE)[None, :]).astype(jnp.int32)
    rank_all = jnp.cumsum(oh, axis=0) - oh
    rank = jnp.take_along_axis(rank_all, e_flat[:, None], axis=1)[:, 0]
    counts = jnp.sum(oh, axis=0)
    pc = ((counts + BLK - 1) // BLK) * BLK
    pbase = jnp.concatenate([jnp.zeros((1,), jnp.int32),
                             jnp.cumsum(pc)[:-1].astype(jnp.int32)])
    padded_pos = pbase[e_flat] + rank
    src_token = jnp.arange(2 * S, dtype=jnp.int32) // 2
    gidx = jnp.zeros((RPAD,), jnp.int32).at[padded_pos].set(src_token)
    w_flat = jnp.take_along_axis(gw8, e_flat.reshape(S, 2), axis=1).reshape(-1)
    ws = jnp.zeros((RPAD,), jnp.float32).at[padded_pos].set(w_flat)
    p0 = padded_pos[0::2]
    p1 = padded_pos[1::2]
    nb_e = pc // BLK
    cumnb = jnp.cumsum(nb_e)
    be = jnp.minimum(
        jnp.searchsorted(cumnb, jnp.arange(NB), side="right"), NE - 1
    ).astype(jnp.int32)

    xg = jnp.take(x2d, gidx, axis=0)

    yg = _grouped_ffn(xg, W1, b1, W2, b2, ws.reshape(RPAD, 1), be)

    out = jnp.take(yg, p0, axis=0) + jnp.take(yg, p1, axis=0)
    return (out[None], gw8[None])
